# baseline (speedup 1.0000x reference)
import functools

import jax
import jax.numpy as jnp
from jax import lax
from jax.experimental import pallas as pl
from jax.experimental.pallas import tpu as pltpu
from jax.experimental.pallas import tpu_sc as plsc

DIM = 1024
BATCH = 16384
NUM_CORES = 2
NUM_SUBCORES = 16
NUM_WORKERS = NUM_CORES * NUM_SUBCORES
ROWS_PER_WORKER = BATCH // NUM_WORKERS
CHUNK = 16
PIECES = 8
PER = CHUNK * PIECES
NUM_CHUNKS = ROWS_PER_WORKER // CHUNK
NBUF = 6
VROWS_PER_WORKER = ROWS_PER_WORKER * PIECES


def _gather_body(idx_hbm, table_hbm, out_hbm, idx_v, *rest):
    rows = rest[:NBUF]
    gsems = rest[NBUF:2 * NBUF]
    ssems = rest[2 * NBUF:3 * NBUF]
    wid = lax.axis_index("s") * NUM_CORES + lax.axis_index("c")
    vbase = wid * VROWS_PER_WORKER
    pltpu.sync_copy(idx_hbm.at[wid], idx_v)

    pending_gather = [None] * NBUF
    pending_store = [None] * NBUF

    def start_gather(c):
        b = c % NBUF
        if pending_store[b] is not None:
            pending_store[b].wait()
            pending_store[b] = None
        g = pltpu.make_async_copy(table_hbm.at[idx_v.at[c]], rows[b], gsems[b])
        g.start()
        pending_gather[b] = g

    def finish_chunk(c):
        b = c % NBUF
        pending_gather[b].wait()
        pending_gather[b] = None
        s = pltpu.make_async_copy(
            rows[b], out_hbm.at[pl.ds(vbase + c * PER, PER)], ssems[b])
        s.start()
        pending_store[b] = s

    for c in range(NBUF - 1):
        start_gather(c)
    for c in range(NBUF - 1, NUM_CHUNKS):
        start_gather(c)
        finish_chunk(c - (NBUF - 1))
    for c in range(NUM_CHUNKS - (NBUF - 1), NUM_CHUNKS):
        finish_chunk(c)
    for p in pending_store:
        if p is not None:
            p.wait()


@jax.jit
def _gather(idx, table_view):
    mesh = plsc.VectorSubcoreMesh(core_axis_name="c", subcore_axis_name="s")
    return pl.kernel(
        _gather_body,
        out_type=jax.ShapeDtypeStruct((BATCH * PIECES, 128), jnp.float32),
        mesh=mesh,
        scratch_types=(
            [pltpu.VMEM((NUM_CHUNKS, PER), jnp.int32)]
            + [pltpu.VMEM((PER, 128), jnp.float32)] * NBUF
            + [pltpu.SemaphoreType.DMA] * (2 * NBUF)
        ),
    )(idx, table_view)


def kernel(input_ids, combined_embedding_weight):
    ids = input_ids.astype(jnp.int32)
    idx8 = ids[:, None] * 8 + jnp.arange(PIECES, dtype=jnp.int32)
    idx8 = idx8.reshape(NUM_WORKERS, NUM_CHUNKS, PER)
    table_view = combined_embedding_weight.reshape(30720 * PIECES, 128)
    out = _gather(idx8, table_view)
    return out.reshape(BATCH, DIM, 1, 1)


# speedup vs baseline: 2.8628x; 2.8628x over previous
import functools

import jax
import jax.numpy as jnp
from jax import lax
from jax.experimental import pallas as pl
from jax.experimental.pallas import tpu as pltpu
from jax.experimental.pallas import tpu_sc as plsc

DIM = 1024
BATCH = 16384
NUM_CORES = 2
NUM_SUBCORES = 16
NUM_WORKERS = NUM_CORES * NUM_SUBCORES
ROWS_PER_WORKER = BATCH // NUM_WORKERS
CHUNK = 16
PIECES = 8
PER = CHUNK * PIECES
NUM_CHUNKS = ROWS_PER_WORKER // CHUNK
NBUF = 6
VROWS_PER_WORKER = ROWS_PER_WORKER * PIECES


def _gather_body(idx_hbm, table_hbm, out_hbm, idx_v, *rest):
    rows = rest[:NBUF]
    gsems = rest[NBUF:2 * NBUF]
    ssems = rest[2 * NBUF:3 * NBUF]
    wid = lax.axis_index("s") * NUM_CORES + lax.axis_index("c")
    vbase = wid * VROWS_PER_WORKER
    pltpu.sync_copy(idx_hbm.at[wid], idx_v)

    pending_gather = [None] * NBUF
    pending_store = [None] * NBUF

    def start_gather(c):
        b = c % NBUF
        if pending_store[b] is not None:
            pending_store[b].wait()
            pending_store[b] = None
        g = pltpu.make_async_copy(table_hbm.at[idx_v.at[c]], rows[b], gsems[b])
        g.start()
        pending_gather[b] = g

    def finish_chunk(c):
        b = c % NBUF
        pending_gather[b].wait()
        pending_gather[b] = None
        s = pltpu.make_async_copy(
            rows[b], out_hbm.at[pl.ds(vbase + c * PER, PER)], ssems[b])
        s.start()
        pending_store[b] = s

    for c in range(NBUF - 1):
        start_gather(c)
    for c in range(NBUF - 1, NUM_CHUNKS):
        start_gather(c)
        finish_chunk(c - (NBUF - 1))
    for c in range(NUM_CHUNKS - (NBUF - 1), NUM_CHUNKS):
        finish_chunk(c)
    for p in pending_store:
        if p is not None:
            p.wait()


@jax.jit
def _gather(idx, table_view):
    mesh = plsc.VectorSubcoreMesh(core_axis_name="c", subcore_axis_name="s")
    return pl.kernel(
        _gather_body,
        out_type=jax.ShapeDtypeStruct((BATCH * PIECES, 128), jnp.float32),
        mesh=mesh,
        scratch_types=(
            [pltpu.VMEM((NUM_CHUNKS, PER), jnp.int32)]
            + [pltpu.VMEM((PER, 128), jnp.float32)] * NBUF
            + [pltpu.SemaphoreType.DMA] * (2 * NBUF)
        ),
    )(idx, table_view)


def kernel(input_ids, combined_embedding_weight):
    ids = input_ids.astype(jnp.int32)
    idx8 = ((ids // 8) * 64 + ids % 8)[:, None] + 8 * jnp.arange(
        PIECES, dtype=jnp.int32)
    idx8 = idx8.reshape(NUM_WORKERS, NUM_CHUNKS, PER)
    table_view = (
        combined_embedding_weight.reshape(3840, 8, PIECES, 128)
        .transpose(0, 2, 1, 3)
        .reshape(30720 * PIECES, 128))
    out = _gather(idx8, table_view)
    return out.reshape(BATCH, DIM, 1, 1)


# speedup vs baseline: 2.8841x; 1.0075x over previous
import functools

import jax
import jax.numpy as jnp
from jax import lax
from jax.experimental import pallas as pl
from jax.experimental.pallas import tpu as pltpu
from jax.experimental.pallas import tpu_sc as plsc

DIM = 1024
BATCH = 16384
TABLE_ROWS = 30720
NUM_CORES = 2
NUM_SUBCORES = 16
NUM_WORKERS = NUM_CORES * NUM_SUBCORES
ROWS_PER_WORKER = BATCH // NUM_WORKERS
CHUNK = 16
PIECES = 8
PER = CHUNK * PIECES
NUM_CHUNKS = ROWS_PER_WORKER // CHUNK
NBUF = 6
LANES = 16


def _gather_body(ids_hbm, table_hbm, out_hbm, ids_v, idx_v, *rest):
    rows = rest[:NBUF]
    gsems = rest[NBUF:2 * NBUF]
    ssems = rest[2 * NBUF:3 * NBUF]
    wid = lax.axis_index("s") * NUM_CORES + lax.axis_index("c")
    base = wid * ROWS_PER_WORKER
    pltpu.sync_copy(ids_hbm.at[pl.ds(base, ROWS_PER_WORKER)], ids_v)

    lane = lax.iota(jnp.int32, LANES)
    for c in range(NUM_CHUNKS):
        idv = ids_v[pl.ds(c * CHUNK, LANES)]
        v0 = ((idv >> 3) << 6) | (idv & 7)
        for k in range(PIECES):
            pos = lane * PIECES + (c * PER + k)
            plsc.store_scatter(idx_v, [pos], v0 + 8 * k)

    pending_gather = [None] * NBUF
    pending_store = [None] * NBUF

    def start_gather(c):
        b = c % NBUF
        if pending_store[b] is not None:
            pending_store[b].wait()
            pending_store[b] = None
        g = pltpu.make_async_copy(
            table_hbm.at[idx_v.at[pl.ds(c * PER, PER)]], rows[b], gsems[b])
        g.start()
        pending_gather[b] = g

    def finish_chunk(c):
        b = c % NBUF
        pending_gather[b].wait()
        pending_gather[b] = None
        s = pltpu.make_async_copy(
            rows[b], out_hbm.at[pl.ds((base + c * CHUNK) * PIECES, PER)],
            ssems[b])
        s.start()
        pending_store[b] = s

    for c in range(NBUF - 1):
        start_gather(c)
    for c in range(NBUF - 1, NUM_CHUNKS):
        start_gather(c)
        finish_chunk(c - (NBUF - 1))
    for c in range(NUM_CHUNKS - (NBUF - 1), NUM_CHUNKS):
        finish_chunk(c)
    for p in pending_store:
        if p is not None:
            p.wait()


@jax.jit
def _gather(ids, table_view):
    mesh = plsc.VectorSubcoreMesh(core_axis_name="c", subcore_axis_name="s")
    return pl.kernel(
        _gather_body,
        out_type=jax.ShapeDtypeStruct((BATCH * PIECES, 128), jnp.float32),
        mesh=mesh,
        compiler_params=pltpu.CompilerParams(needs_layout_passes=False),
        scratch_types=(
            [pltpu.VMEM((ROWS_PER_WORKER,), jnp.int32),
             pltpu.VMEM((NUM_CHUNKS * PER,), jnp.int32)]
            + [pltpu.VMEM((PER, 128), jnp.float32)] * NBUF
            + [pltpu.SemaphoreType.DMA] * (2 * NBUF)
        ),
    )(ids, table_view)


def kernel(input_ids, combined_embedding_weight):
    ids = input_ids.astype(jnp.int32)
    table_view = (
        combined_embedding_weight.reshape(3840, 8, PIECES, 128)
        .transpose(0, 2, 1, 3)
        .reshape(TABLE_ROWS * PIECES, 128))
    out = _gather(ids, table_view)
    return out.reshape(BATCH, DIM, 1, 1)
